# Initial kernel scaffold; baseline (speedup 1.0000x reference)
import jax
import jax.numpy as jnp
from jax.experimental import pallas as pl


def kernel(xs, W_att, b_att):
    raise NotImplementedError("write your pallas kernel here")


# speedup vs baseline: 22.5661x; 22.5661x over previous
import jax
import jax.numpy as jnp
from jax.experimental import pallas as pl


_BLK = 2048


def _merge_blk(xs_ref, w_ref, b_ref, emb_ref, s_ref):
    d = xs_ref.shape[-1]
    w = w_ref[:, 0]
    w_msg = w[:d]
    w_q = w[d:]
    b = b_ref[0]

    def nrm(x):
        n = jnp.sqrt(jnp.sum(x * x, axis=-1, keepdims=True))
        return x / jnp.maximum(n, 1e-12)

    q = nrm(xs_ref[0])
    m1 = nrm(xs_ref[1])
    m2 = nrm(xs_ref[2])
    m3 = nrm(xs_ref[3])

    qdot = jnp.sum(q * w_q[None, :], axis=-1) + b

    def score(m):
        s = jnp.sum(m * w_msg[None, :], axis=-1) + qdot
        return jnp.where(s >= 0, s, 0.01 * s)

    s1 = score(m1)
    s2 = score(m2)
    s3 = score(m3)
    smax = jnp.maximum(jnp.maximum(s1, s2), s3)
    e1 = jnp.exp(s1 - smax)
    e2 = jnp.exp(s2 - smax)
    e3 = jnp.exp(s3 - smax)
    inv = 1.0 / (e1 + e2 + e3 + 1e-16)
    a1 = e1 * inv
    a2 = e2 * inv
    a3 = e3 * inv

    emb_ref[...] = q + a1[:, None] * m1 + a2[:, None] * m2 + a3[:, None] * m3
    s_ref[0, :] = a1
    s_ref[1, :] = a2
    s_ref[2, :] = a3


def kernel(xs, W_att, b_att):
    L, N, d = xs.shape
    blk = _BLK
    grid = ((N + blk - 1) // blk,)
    emb, sc = pl.pallas_call(
        _merge_blk,
        grid=grid,
        in_specs=[
            pl.BlockSpec((L, blk, d), lambda i: (0, i, 0)),
            pl.BlockSpec((2 * d, 1), lambda i: (0, 0)),
            pl.BlockSpec((1,), lambda i: (0,)),
        ],
        out_specs=[
            pl.BlockSpec((blk, d), lambda i: (i, 0)),
            pl.BlockSpec((L - 1, blk), lambda i: (0, i)),
        ],
        out_shape=[
            jax.ShapeDtypeStruct((N, d), xs.dtype),
            jax.ShapeDtypeStruct((L - 1, N), xs.dtype),
        ],
    )(xs, W_att, b_att)
    return emb, sc.reshape(-1)


# speedup vs baseline: 33.4249x; 1.4812x over previous
import jax
import jax.numpy as jnp
from jax.experimental import pallas as pl


_BLK = 4096


def _merge_blk(xs_ref, w_ref, b_ref, emb_ref, s_ref):
    d = xs_ref.shape[-1]
    blk = xs_ref.shape[1]
    b = b_ref[0]

    q = xs_ref[0]
    m1 = xs_ref[1]
    m2 = xs_ref[2]
    m3 = xs_ref[3]

    xx = xs_ref[...].reshape(4 * blk, d)
    wb = jnp.concatenate([w_ref[:d, :], w_ref[d:, :]], axis=1)
    ones_col = jnp.ones((d, 1), dtype=xx.dtype)
    ssq = jnp.dot(xx * xx, ones_col, preferred_element_type=jnp.float32)
    dots = jnp.dot(xx, wb, preferred_element_type=jnp.float32)

    ssq_t = ssq.T
    dots_t = dots.T

    inv_all = jax.lax.rsqrt(jnp.maximum(ssq_t, 1e-24))
    iq = inv_all[:, 0:blk]
    i1 = inv_all[:, blk : 2 * blk]
    i2 = inv_all[:, 2 * blk : 3 * blk]
    i3 = inv_all[:, 3 * blk : 4 * blk]

    qterm = dots_t[1:2, 0:blk] * iq + b

    def score(dt, inv):
        s = dt * inv + qterm
        return jnp.where(s >= 0, s, 0.01 * s)

    s1 = score(dots_t[0:1, blk : 2 * blk], i1)
    s2 = score(dots_t[0:1, 2 * blk : 3 * blk], i2)
    s3 = score(dots_t[0:1, 3 * blk : 4 * blk], i3)
    smax = jnp.maximum(jnp.maximum(s1, s2), s3)
    e1 = jnp.exp(s1 - smax)
    e2 = jnp.exp(s2 - smax)
    e3 = jnp.exp(s3 - smax)
    r = 1.0 / (e1 + e2 + e3 + 1e-16)
    a1 = e1 * r
    a2 = e2 * r
    a3 = e3 * r

    s_ref[0:1, :] = a1
    s_ref[1:2, :] = a2
    s_ref[2:3, :] = a3

    cm = jnp.concatenate([iq, a1 * i1, a2 * i2, a3 * i3], axis=0)
    ct = cm.T
    emb_ref[...] = (
        ct[:, 0:1] * q + ct[:, 1:2] * m1 + ct[:, 2:3] * m2 + ct[:, 3:4] * m3
    )


def kernel(xs, W_att, b_att):
    L, N, d = xs.shape
    blk = _BLK
    grid = ((N + blk - 1) // blk,)
    emb, sc = pl.pallas_call(
        _merge_blk,
        grid=grid,
        in_specs=[
            pl.BlockSpec((L, blk, d), lambda i: (0, i, 0)),
            pl.BlockSpec((2 * d, 1), lambda i: (0, 0)),
            pl.BlockSpec((1,), lambda i: (0,)),
        ],
        out_specs=[
            pl.BlockSpec((blk, d), lambda i: (i, 0)),
            pl.BlockSpec((L - 1, blk), lambda i: (0, i)),
        ],
        out_shape=[
            jax.ShapeDtypeStruct((N, d), xs.dtype),
            jax.ShapeDtypeStruct((L - 1, N), xs.dtype),
        ],
    )(xs, W_att, b_att)
    return emb, sc.reshape(-1)


# speedup vs baseline: 60.6923x; 1.8158x over previous
import jax
import jax.numpy as jnp
from jax.experimental import pallas as pl


_BLK = 6272


def _merge_blk(xs_ref, w_ref, b_ref, emb_ref, s_ref):
    d = xs_ref.shape[-1]
    blk = xs_ref.shape[1]
    b = b_ref[0]

    q = xs_ref[0]
    m1 = xs_ref[1]
    m2 = xs_ref[2]
    m3 = xs_ref[3]

    zcol = jnp.zeros((d, 1), dtype=q.dtype)
    ocol = jnp.ones((d, 1), dtype=q.dtype)
    ones_lo = jnp.concatenate([zcol, ocol], axis=0)
    r_msg = jnp.concatenate(
        [jnp.concatenate([w_ref[:d, :], zcol], axis=0), ones_lo], axis=1
    )
    r_q = jnp.concatenate(
        [jnp.concatenate([w_ref[d:, :], zcol], axis=0), ones_lo], axis=1
    )

    def red(x, r):
        aug = jnp.concatenate([x, x * x], axis=1)
        out = jnp.dot(aug, r, preferred_element_type=jnp.float32)
        out_t = out.T
        return out_t[1:2, :], out_t[0:1, :]

    ssq_q, dot_q = red(q, r_q)
    ssq_1, dot_1 = red(m1, r_msg)
    ssq_2, dot_2 = red(m2, r_msg)
    ssq_3, dot_3 = red(m3, r_msg)

    def inv_norm(ssq):
        return jax.lax.rsqrt(jnp.maximum(ssq, 1e-24))

    iq = inv_norm(ssq_q)
    i1 = inv_norm(ssq_1)
    i2 = inv_norm(ssq_2)
    i3 = inv_norm(ssq_3)

    qterm = dot_q * iq + b

    def score(dt, inv):
        s = dt * inv + qterm
        return jnp.where(s >= 0, s, 0.01 * s)

    s1 = score(dot_1, i1)
    s2 = score(dot_2, i2)
    s3 = score(dot_3, i3)
    smax = jnp.maximum(jnp.maximum(s1, s2), s3)
    e1 = jnp.exp(s1 - smax)
    e2 = jnp.exp(s2 - smax)
    e3 = jnp.exp(s3 - smax)
    r = 1.0 / (e1 + e2 + e3 + 1e-16)
    a1 = e1 * r
    a2 = e2 * r
    a3 = e3 * r

    s_ref[0:1, :] = a1
    s_ref[1:2, :] = a2
    s_ref[2:3, :] = a3

    cm = jnp.concatenate([iq, a1 * i1, a2 * i2, a3 * i3], axis=0)
    ct = cm.T
    lane = jax.lax.broadcasted_iota(jnp.int32, (4, 4 * d), 1) // d
    row = jax.lax.broadcasted_iota(jnp.int32, (4, 4 * d), 0)
    one_map = (lane == row).astype(q.dtype)
    bc = jnp.dot(ct, one_map, preferred_element_type=jnp.float32)
    emb_ref[...] = (
        bc[:, 0:d] * q
        + bc[:, d : 2 * d] * m1
        + bc[:, 2 * d : 3 * d] * m2
        + bc[:, 3 * d : 4 * d] * m3
    )


def kernel(xs, W_att, b_att):
    L, N, d = xs.shape
    blk = _BLK
    grid = ((N + blk - 1) // blk,)
    emb, sc = pl.pallas_call(
        _merge_blk,
        grid=grid,
        in_specs=[
            pl.BlockSpec((L, blk, d), lambda i: (0, i, 0)),
            pl.BlockSpec((2 * d, 1), lambda i: (0, 0)),
            pl.BlockSpec((1,), lambda i: (0,)),
        ],
        out_specs=[
            pl.BlockSpec((blk, d), lambda i: (i, 0)),
            pl.BlockSpec((L - 1, blk), lambda i: (0, i)),
        ],
        out_shape=[
            jax.ShapeDtypeStruct((N, d), xs.dtype),
            jax.ShapeDtypeStruct((L - 1, N), xs.dtype),
        ],
    )(xs, W_att, b_att)
    return emb, sc.reshape(-1)
